# baseline (speedup 1.0000x reference)
import functools

import numpy as np
import jax
import jax.numpy as jnp
from jax.experimental import pallas as pl

_NUM_BINS = 15
_BOUNDS = np.linspace(0.0, 1.0, _NUM_BINS + 1, dtype=np.float32)
_LO128 = np.full((128,), 2.0, dtype=np.float32)
_UP128 = np.full((128,), 3.0, dtype=np.float32)
_LO128[:_NUM_BINS] = _BOUNDS[:-1]
_UP128[:_NUM_BINS] = _BOUNDS[1:]


_BND8 = np.zeros((8, 128), dtype=np.float32)
_BND8[0, :] = _LO128
_BND8[1, :] = _UP128


def _ece_body(x_ref, lbl_ref, bnd_ref, out_ref, *, nsteps, n_total, ncls):
    i = pl.program_id(0)

    @pl.when(i == 0)
    def _init():
        out_ref[...] = jnp.zeros_like(out_ref)

    x = x_ref[...]
    lbl = lbl_ref[0, 0, :]
    m = jnp.max(x, axis=1)
    cls_iota = jax.lax.broadcasted_iota(jnp.int32, x.shape, 1)
    first = jnp.min(jnp.where(x == m[:, None], cls_iota, ncls), axis=1)
    acc = (first == lbl).astype(jnp.float32)
    s = jnp.sum(jnp.exp(x - m[:, None]), axis=1)
    conf = 1.0 / s

    lo = bnd_ref[0:1, :]
    up = bnd_ref[1:2, :]
    msk = ((conf[:, None] > lo) & (conf[:, None] <= up)
           ).astype(jnp.float32)
    out_ref[0:1, :] += jnp.sum(msk, axis=0, keepdims=True)
    out_ref[1:2, :] += jnp.sum(msk * conf[:, None], axis=0, keepdims=True)
    out_ref[2:3, :] += jnp.sum(msk * acc[:, None], axis=0, keepdims=True)

    @pl.when(i == nsteps - 1)
    def _finish():
        cnt = out_ref[0:1, :]
        sc = out_ref[1:2, :]
        sa = out_ref[2:3, :]
        denom = jnp.maximum(cnt, 1.0)
        prop = cnt / n_total
        per = jnp.where(cnt > 0, jnp.abs(sc / denom - sa / denom) * prop, 0.0)
        out_ref[3:4, :] = jnp.broadcast_to(jnp.sum(per), (1, 128))


def kernel(logits, labels):
    n, c = logits.shape
    blk = 4000
    nsteps = n // blk
    lbl3 = labels.astype(jnp.int32).reshape(nsteps, 1, blk)
    out = pl.pallas_call(
        functools.partial(_ece_body, nsteps=nsteps, n_total=float(n), ncls=c),
        grid=(nsteps,),
        in_specs=[
            pl.BlockSpec((blk, c), lambda i: (i, 0)),
            pl.BlockSpec((1, 1, blk), lambda i: (i, 0, 0)),
            pl.BlockSpec((8, 128), lambda i: (0, 0)),
        ],
        out_specs=pl.BlockSpec((8, 128), lambda i: (0, 0)),
        out_shape=jax.ShapeDtypeStruct((8, 128), jnp.float32),
    )(logits, lbl3, jnp.asarray(_BND8))
    return out[3, 0].reshape(1)


# speedup vs baseline: 1.1792x; 1.1792x over previous
import functools

import numpy as np
import jax
import jax.numpy as jnp
from jax.experimental import pallas as pl

_NUM_BINS = 15
_BOUNDS = np.linspace(0.0, 1.0, _NUM_BINS + 1, dtype=np.float32)
_LO128 = np.full((128,), 2.0, dtype=np.float32)
_UP128 = np.full((128,), 3.0, dtype=np.float32)
_LO128[:_NUM_BINS] = _BOUNDS[:-1]
_UP128[:_NUM_BINS] = _BOUNDS[1:]


_BND8 = np.zeros((8, 128), dtype=np.float32)
_BND8[0, :] = _LO128
_BND8[1, :] = _UP128


def _ece_body(x_ref, lbl_ref, bnd_ref, out_ref, *, nsteps, n_total, ncls):
    i = pl.program_id(0)

    @pl.when(i == 0)
    def _init():
        out_ref[...] = jnp.zeros_like(out_ref)

    x = x_ref[...]
    lbl = lbl_ref[0, 0, :]
    m = jnp.max(x, axis=1)
    cls_iota = jax.lax.broadcasted_iota(jnp.int32, x.shape, 1).astype(jnp.float32)
    first = jnp.min(jnp.where(x == m[:, None], cls_iota, float(ncls)), axis=1)
    acc = (first == lbl).astype(jnp.float32)
    s = jnp.sum(jnp.exp(x - m[:, None]), axis=1)
    conf = 1.0 / s

    lo = bnd_ref[0:1, :]
    up = bnd_ref[1:2, :]
    msk = ((conf[:, None] > lo) & (conf[:, None] <= up)
           ).astype(jnp.float32)
    out_ref[0:1, :] += jnp.sum(msk, axis=0, keepdims=True)
    out_ref[1:2, :] += jnp.sum(msk * conf[:, None], axis=0, keepdims=True)
    out_ref[2:3, :] += jnp.sum(msk * acc[:, None], axis=0, keepdims=True)

    @pl.when(i == nsteps - 1)
    def _finish():
        cnt = out_ref[0:1, :]
        sc = out_ref[1:2, :]
        sa = out_ref[2:3, :]
        denom = jnp.maximum(cnt, 1.0)
        prop = cnt / n_total
        per = jnp.where(cnt > 0, jnp.abs(sc / denom - sa / denom) * prop, 0.0)
        out_ref[3:4, :] = jnp.broadcast_to(jnp.sum(per), (1, 128))


def kernel(logits, labels):
    n, c = logits.shape
    blk = 4000
    nsteps = n // blk
    lbl3 = labels.astype(jnp.float32).reshape(nsteps, 1, blk)
    out = pl.pallas_call(
        functools.partial(_ece_body, nsteps=nsteps, n_total=float(n), ncls=c),
        grid=(nsteps,),
        in_specs=[
            pl.BlockSpec((blk, c), lambda i: (i, 0)),
            pl.BlockSpec((1, 1, blk), lambda i: (i, 0, 0)),
            pl.BlockSpec((8, 128), lambda i: (0, 0)),
        ],
        out_specs=pl.BlockSpec((8, 128), lambda i: (0, 0)),
        out_shape=jax.ShapeDtypeStruct((8, 128), jnp.float32),
    )(logits, lbl3, jnp.asarray(_BND8))
    return out[3, 0].reshape(1)


# speedup vs baseline: 1.9271x; 1.6342x over previous
import functools

import numpy as np
import jax
import jax.numpy as jnp
from jax.experimental import pallas as pl

_NUM_BINS = 15
_LOG2E = float(np.log2(np.e))
_BOUNDS = np.linspace(0.0, 1.0, _NUM_BINS + 1, dtype=np.float32)
_BND = np.zeros((16, 128), dtype=np.float32)
_BND[:, 0] = 2.0
_BND[:, 1] = 3.0
_BND[:_NUM_BINS, 0] = _BOUNDS[:-1]
_BND[:_NUM_BINS, 1] = _BOUNDS[1:]


def _ece_body(x_ref, lbl_ref, bnd_ref, out_ref, *, nsteps, n_total, ncls):
    i = pl.program_id(0)

    @pl.when(i == 0)
    def _init():
        out_ref[...] = jnp.zeros_like(out_ref)

    xt = jnp.transpose(x_ref[...])
    lbl = lbl_ref[0, :, :]

    e = jnp.exp2(xt * _LOG2E)
    s = jnp.sum(e, axis=0, keepdims=True)
    emax = jnp.max(e, axis=0, keepdims=True)
    row_iota = jax.lax.broadcasted_iota(jnp.int32, xt.shape, 0)
    e_lbl = jnp.max(jnp.where(row_iota == lbl, e, 0.0), axis=0, keepdims=True)

    conf = emax / s
    accb = e_lbl == emax
    accf = jnp.where(accb, 1.0, 0.0)

    lo = bnd_ref[:, 0:1]
    up = bnd_ref[:, 1:2]
    ohb = (conf > lo) & (conf <= up)
    cnt = jnp.sum(jnp.where(ohb, 1.0, 0.0), axis=1, keepdims=True)
    sc = jnp.sum(jnp.where(ohb, conf, 0.0), axis=1, keepdims=True)
    sa = jnp.sum(jnp.where(ohb, accf, 0.0), axis=1, keepdims=True)

    out_ref[0:16, 0:1] += cnt
    out_ref[0:16, 1:2] += sc
    out_ref[0:16, 2:3] += sa

    @pl.when(i == nsteps - 1)
    def _finish():
        cntv = out_ref[0:16, 0:1]
        scv = out_ref[0:16, 1:2]
        sav = out_ref[0:16, 2:3]
        denom = jnp.maximum(cntv, 1.0)
        prop = cntv / n_total
        per = jnp.where(cntv > 0,
                        jnp.abs(scv / denom - sav / denom) * prop, 0.0)
        out_ref[0:16, 3:4] = jnp.broadcast_to(jnp.sum(per), (16, 1))


def kernel(logits, labels):
    n, c = logits.shape
    blk = 4000
    nsteps = n // blk
    lbl3 = labels.astype(jnp.int32).reshape(nsteps, 1, blk)
    out = pl.pallas_call(
        functools.partial(_ece_body, nsteps=nsteps, n_total=float(n), ncls=c),
        grid=(nsteps,),
        in_specs=[
            pl.BlockSpec((blk, c), lambda i: (i, 0)),
            pl.BlockSpec((1, 1, blk), lambda i: (i, 0, 0)),
            pl.BlockSpec((16, 128), lambda i: (0, 0)),
        ],
        out_specs=pl.BlockSpec((16, 128), lambda i: (0, 0)),
        out_shape=jax.ShapeDtypeStruct((16, 128), jnp.float32),
    )(logits, lbl3, jnp.asarray(_BND))
    return out[0, 3].reshape(1)


# speedup vs baseline: 2.3854x; 1.2378x over previous
import functools

import numpy as np
import jax
import jax.numpy as jnp
from jax.experimental import pallas as pl

_NUM_BINS = 15
_LOG2E = float(np.log2(np.e))
_BOUNDS = np.linspace(0.0, 1.0, _NUM_BINS + 1, dtype=np.float32)
_BND = np.zeros((16, 128), dtype=np.float32)
_BND[:, 0] = 2.0
_BND[:, 1] = 3.0
_BND[:_NUM_BINS, 0] = _BOUNDS[:-1]
_BND[:_NUM_BINS, 1] = _BOUNDS[1:]


def _ece_body(x_ref, lbl_ref, bnd_ref, out_ref, *, nsteps, n_total, ncls):
    i = pl.program_id(0)

    @pl.when(i == 0)
    def _init():
        out_ref[...] = jnp.zeros_like(out_ref)

    xt = jnp.transpose(x_ref[...])
    lbl = lbl_ref[0, :, :]

    e = jnp.exp2(xt * _LOG2E)
    s = jnp.sum(e, axis=0, keepdims=True)
    emax = jnp.max(e, axis=0, keepdims=True)
    row_iota = jax.lax.broadcasted_iota(jnp.int32, xt.shape, 0)
    e_lbl = jnp.max(jnp.where(row_iota == lbl, e, 0.0), axis=0, keepdims=True)

    conf = emax / s
    accb = e_lbl == emax
    accf = jnp.where(accb, 1.0, 0.0)

    lo = bnd_ref[:, 0:1]
    up = bnd_ref[:, 1:2]
    ohb = (conf > lo) & (conf <= up)
    cnt = jnp.sum(jnp.where(ohb, 1.0, 0.0), axis=1, keepdims=True)
    sc = jnp.sum(jnp.where(ohb, conf, 0.0), axis=1, keepdims=True)
    sa = jnp.sum(jnp.where(ohb, accf, 0.0), axis=1, keepdims=True)

    out_ref[0:16, 0:1] += cnt
    out_ref[0:16, 1:2] += sc
    out_ref[0:16, 2:3] += sa

    @pl.when(i == nsteps - 1)
    def _finish():
        cntv = out_ref[0:16, 0:1]
        scv = out_ref[0:16, 1:2]
        sav = out_ref[0:16, 2:3]
        denom = jnp.maximum(cntv, 1.0)
        prop = cntv / n_total
        per = jnp.where(cntv > 0,
                        jnp.abs(scv / denom - sav / denom) * prop, 0.0)
        out_ref[0:16, 3:4] = jnp.broadcast_to(jnp.sum(per), (16, 1))


def kernel(logits, labels):
    n, c = logits.shape
    blk = 20000
    nsteps = n // blk
    lbl3 = labels.astype(jnp.int32).reshape(nsteps, 1, blk)
    out = pl.pallas_call(
        functools.partial(_ece_body, nsteps=nsteps, n_total=float(n), ncls=c),
        grid=(nsteps,),
        in_specs=[
            pl.BlockSpec((blk, c), lambda i: (i, 0)),
            pl.BlockSpec((1, 1, blk), lambda i: (i, 0, 0)),
            pl.BlockSpec((16, 128), lambda i: (0, 0)),
        ],
        out_specs=pl.BlockSpec((16, 128), lambda i: (0, 0)),
        out_shape=jax.ShapeDtypeStruct((16, 128), jnp.float32),
    )(logits, lbl3, jnp.asarray(_BND))
    return out[0, 3].reshape(1)


# speedup vs baseline: 4.5391x; 1.9029x over previous
import functools

import numpy as np
import jax
import jax.numpy as jnp
from jax.experimental import pallas as pl

_NUM_BINS = 15
_LOG2E = float(np.log2(np.e))
_BOUNDS = np.linspace(0.0, 1.0, _NUM_BINS + 1, dtype=np.float32)
_BND = np.zeros((16, 128), dtype=np.float32)
_BND[:, 0] = 2.0
_BND[:, 1] = 3.0
_BND[:_NUM_BINS, 0] = _BOUNDS[:-1]
_BND[:_NUM_BINS, 1] = _BOUNDS[1:]


def _ece_body(x_ref, lbl_ref, bnd_ref, out_ref, *, nsteps, n_total, ncls, blk):
    i = pl.program_id(0)

    @pl.when(i == 0)
    def _init():
        out_ref[...] = jnp.zeros_like(out_ref)

    xt = x_ref[...]
    lbl = lbl_ref[0, :, :]

    e = jnp.exp2(xt * _LOG2E)
    s = jnp.sum(e, axis=0, keepdims=True)
    emax = jnp.max(e, axis=0, keepdims=True)
    row_iota = jax.lax.broadcasted_iota(jnp.int32, xt.shape, 0)
    e_lbl = jnp.max(jnp.where(row_iota == lbl, e, 0.0), axis=0, keepdims=True)

    conf = emax / s
    accb = e_lbl == emax
    accf = jnp.where(accb, 1.0, 0.0)

    lo = bnd_ref[:, 0:1]
    up = bnd_ref[:, 1:2]
    lane = jax.lax.broadcasted_iota(jnp.int32, conf.shape, 1)
    valid = lane < (jnp.int32(n_total) - i * blk)
    ohb = (conf > lo) & (conf <= up) & valid
    cnt = jnp.sum(jnp.where(ohb, 1.0, 0.0), axis=1, keepdims=True)
    sc = jnp.sum(jnp.where(ohb, conf, 0.0), axis=1, keepdims=True)
    sa = jnp.sum(jnp.where(ohb, accf, 0.0), axis=1, keepdims=True)

    out_ref[0:16, 0:1] += cnt
    out_ref[0:16, 1:2] += sc
    out_ref[0:16, 2:3] += sa

    @pl.when(i == nsteps - 1)
    def _finish():
        cntv = out_ref[0:16, 0:1]
        scv = out_ref[0:16, 1:2]
        sav = out_ref[0:16, 2:3]
        denom = jnp.maximum(cntv, 1.0)
        prop = cntv / n_total
        per = jnp.where(cntv > 0,
                        jnp.abs(scv / denom - sav / denom) * prop, 0.0)
        out_ref[0:16, 3:4] = jnp.broadcast_to(jnp.sum(per), (16, 1))


def kernel(logits, labels):
    n, c = logits.shape
    blk = 16384
    nsteps = (n + blk - 1) // blk
    cp = (c + 7) // 8 * 8
    xt_full = jnp.transpose(
        jnp.pad(logits, ((0, 0), (0, cp - c)), constant_values=-1e30)
    )
    lbl3 = jnp.pad(labels.astype(jnp.int32),
                   (0, nsteps * blk - n)).reshape(nsteps, 1, blk)
    out = pl.pallas_call(
        functools.partial(_ece_body, nsteps=nsteps, n_total=float(n), ncls=c,
                          blk=blk),
        grid=(nsteps,),
        in_specs=[
            pl.BlockSpec((cp, blk), lambda i: (0, i)),
            pl.BlockSpec((1, 1, blk), lambda i: (i, 0, 0)),
            pl.BlockSpec((16, 128), lambda i: (0, 0)),
        ],
        out_specs=pl.BlockSpec((16, 128), lambda i: (0, 0)),
        out_shape=jax.ShapeDtypeStruct((16, 128), jnp.float32),
    )(xt_full, lbl3, jnp.asarray(_BND))
    return out[0, 3].reshape(1)
